# Initial kernel scaffold; baseline (speedup 1.0000x reference)
import jax
import jax.numpy as jnp
from jax.experimental import pallas as pl


def kernel(indices, element_atomic_numbers, cgcnn_table):
    raise NotImplementedError("write your pallas kernel here")


# speedup vs baseline: 2.7485x; 2.7485x over previous
import functools

import jax
import jax.numpy as jnp
from jax import lax
from jax.experimental import pallas as pl
from jax.experimental.pallas import tpu as pltpu
from jax.experimental.pallas import tpu_sc as plsc

N = 100000
D = 92
DP = 96
EAN_LEN = 89
EAN_PAD = 96
MACRO = 512
GRP = 128
NMACRO = -(-N // MACRO)
NPAD = NMACRO * MACRO
TAIL_ROWS = N - (NMACRO - 1) * MACRO

_info = plsc.get_sparse_core_info()
_NC, _NS, _L = _info.num_cores, _info.num_subcores, _info.num_lanes
NW = _NC * _NS
ITERS = -(-NMACRO // NW)

_mesh = plsc.VectorSubcoreMesh(core_axis_name="c", subcore_axis_name="s")


@functools.partial(
    pl.kernel,
    mesh=_mesh,
    out_type=jax.ShapeDtypeStruct((N * D,), jnp.float32),
    compiler_params=pltpu.CompilerParams(
        needs_layout_passes=False, use_tc_tiling_on_sc=False),
    scratch_types=[
        pltpu.VMEM((EAN_PAD,), jnp.int32),
        pltpu.VMEM((MACRO,), jnp.int32),
        pltpu.VMEM((MACRO,), jnp.int32),
        pltpu.VMEM((MACRO, DP), jnp.float32),
        pltpu.VMEM((MACRO * D + 16,), jnp.float32),
        pltpu.SemaphoreType.DMA,
    ],
)
def _gather_kernel(idx_hbm, ean_hbm, table_hbm, out_hbm,
                   ean_v, idx_v, aidx_v, rows_v, comp_v, sem):
    wid = lax.axis_index("s") * _NC + lax.axis_index("c")
    pltpu.sync_copy(ean_hbm, ean_v)

    def body(it, carry):
        g = wid + it * NW

        @pl.when(g < NMACRO)
        def _():
            base = g * MACRO
            pltpu.sync_copy(idx_hbm.at[pl.ds(base, MACRO)], idx_v)
            for j in range(MACRO // _L):
                v = idx_v[pl.ds(j * _L, _L)]
                aidx_v[pl.ds(j * _L, _L)] = plsc.load_gather(ean_v, [v])
            copies = []
            for k in range(MACRO // GRP):
                copies.append(pltpu.async_copy(
                    table_hbm.at[aidx_v.at[pl.ds(k * GRP, GRP)]],
                    rows_v.at[pl.ds(k * GRP, GRP)],
                    sem,
                ))
            for c in copies:
                c.wait()

            def compact(q, carry2):
                r = q * 4
                for rr in range(4):
                    for j in range(6):
                        src = rows_v[r + rr, pl.ds(j * _L, _L)]
                        comp_v[pl.ds((r + rr) * D + j * _L, _L)] = src
                return carry2

            lax.fori_loop(0, MACRO // 4, compact, 0)

            @pl.when(g < NMACRO - 1)
            def _full():
                pltpu.sync_copy(comp_v.at[pl.ds(0, MACRO * D)],
                                out_hbm.at[pl.ds(base * D, MACRO * D)])

            @pl.when(g == NMACRO - 1)
            def _tail():
                pltpu.sync_copy(comp_v.at[pl.ds(0, TAIL_ROWS * D)],
                                out_hbm.at[pl.ds(base * D, TAIL_ROWS * D)])

        return carry

    lax.fori_loop(0, ITERS, body, 0)


def kernel(indices, element_atomic_numbers, cgcnn_table):
    idx = jnp.asarray(indices, jnp.int32)
    idx = jnp.concatenate([idx, jnp.zeros((NPAD - N,), jnp.int32)])
    ean = jnp.concatenate([
        jnp.asarray(element_atomic_numbers, jnp.int32),
        jnp.zeros((EAN_PAD - EAN_LEN,), jnp.int32),
    ])
    table = jnp.pad(jnp.asarray(cgcnn_table, jnp.float32),
                    ((0, 0), (0, DP - D)))
    return _gather_kernel(idx, ean, table).reshape(N, D)


# speedup vs baseline: 3.1351x; 1.1406x over previous
import functools

import jax
import jax.numpy as jnp
from jax import lax
from jax.experimental import pallas as pl
from jax.experimental.pallas import tpu as pltpu
from jax.experimental.pallas import tpu_sc as plsc

N = 100000
D = 92
PD = 2 * D
NP = N // 2
EAN_LEN = 89
EAN_PAD = 96
MACRO = 200
GRP0 = 128
GRP1 = MACRO - GRP0
NMACRO = NP // MACRO
RGROUPS = -(-MACRO // 16)

_info = plsc.get_sparse_core_info()
_NC, _NS, _L = _info.num_cores, _info.num_subcores, _info.num_lanes
NW = _NC * _NS
ITERS = -(-NMACRO // NW)
NBUF = 2

_mesh = plsc.VectorSubcoreMesh(core_axis_name="c", subcore_axis_name="s")


@functools.partial(
    pl.kernel,
    mesh=_mesh,
    out_type=jax.ShapeDtypeStruct((NP, PD), jnp.float32),
    compiler_params=pltpu.CompilerParams(
        needs_layout_passes=False, use_tc_tiling_on_sc=False),
    scratch_types=[
        pltpu.VMEM((EAN_PAD,), jnp.int32),
        pltpu.VMEM((2 * MACRO,), jnp.int32),
        pltpu.VMEM((16 * RGROUPS,), jnp.int32),
        pltpu.VMEM((NBUF, MACRO, PD), jnp.float32),
        pltpu.SemaphoreType.DMA,
        pltpu.SemaphoreType.DMA,
        pltpu.SemaphoreType.DMA,
    ],
)
def _gather_kernel(idx_hbm, ean_hbm, pairtab_hbm, out_hbm,
                   ean_v, idx_v, pidx_v, rows_v, gsem, ssem0, ssem1):
    wid = lax.axis_index("s") * _NC + lax.axis_index("c")
    pltpu.sync_copy(ean_hbm, ean_v)
    ssems = [ssem0, ssem1]
    lanes = lax.iota(jnp.int32, _L)
    stores = [None] * ITERS

    for it in range(ITERS):
        g = wid + it * NW
        buf = it % NBUF

        if it >= NBUF:
            prev_g = wid + (it - NBUF) * NW

            @pl.when(prev_g < NMACRO)
            def _drain(d=stores[it - NBUF]):
                d.wait()

        @pl.when(g < NMACRO)
        def _chunk():
            base = g * MACRO
            pltpu.sync_copy(idx_hbm.at[pl.ds(2 * base, 2 * MACRO)], idx_v)
            for j in range(RGROUPS):
                pos = jnp.minimum(2 * _L * j + 2 * lanes, 2 * MACRO - 2)
                even = plsc.load_gather(idx_v, [pos])
                odd = plsc.load_gather(idx_v, [pos + 1])
                a = plsc.load_gather(ean_v, [even])
                b = plsc.load_gather(ean_v, [odd])
                pidx_v[pl.ds(j * _L, _L)] = a * 100 + b
            c0 = pltpu.async_copy(
                pairtab_hbm.at[pidx_v.at[pl.ds(0, GRP0)]],
                rows_v.at[buf, pl.ds(0, GRP0)], gsem)
            c1 = pltpu.async_copy(
                pairtab_hbm.at[pidx_v.at[pl.ds(GRP0, GRP1)]],
                rows_v.at[buf, pl.ds(GRP0, GRP1)], gsem)
            c0.wait()
            c1.wait()

        gc = jnp.minimum(g, NMACRO - 1)
        store = pltpu.make_async_copy(
            rows_v.at[buf], out_hbm.at[pl.ds(gc * MACRO, MACRO)], ssems[buf])
        stores[it] = store

        @pl.when(g < NMACRO)
        def _start(d=store):
            d.start()

    for it in range(ITERS - NBUF, ITERS):
        last_g = wid + it * NW

        @pl.when(last_g < NMACRO)
        def _final(d=stores[it]):
            d.wait()


def kernel(indices, element_atomic_numbers, cgcnn_table):
    idx = jnp.asarray(indices, jnp.int32)
    ean = jnp.concatenate([
        jnp.asarray(element_atomic_numbers, jnp.int32),
        jnp.zeros((EAN_PAD - EAN_LEN,), jnp.int32),
    ])
    table = jnp.asarray(cgcnn_table, jnp.float32)
    pairtab = jnp.concatenate([
        jnp.broadcast_to(table[:, None, :], (100, 100, D)),
        jnp.broadcast_to(table[None, :, :], (100, 100, D)),
    ], axis=-1).reshape(100 * 100, PD)
    return _gather_kernel(idx, ean, pairtab).reshape(N, D)
